# baseline (speedup 1.0000x reference)
import jax
import jax.numpy as jnp
import numpy as np
from jax.experimental import pallas as pl


def _fp_kernel(xt_ref, xs_ref, ft_ref, fs_ref, w1t_ref, x1_ref, stats_ref):
    S = xs_ref.shape[1]
    xt = xt_ref[0]
    xs = xs_ref[0]
    ft = ft_ref[0]
    fs = fs_ref[0]
    xt2 = jnp.sum(xt * xt, axis=1, keepdims=True)
    xs2 = jnp.sum(xs * xs, axis=1, keepdims=True)
    dot = jnp.dot(xt, xs.T, preferred_element_type=jnp.float32)
    d = jnp.maximum(xt2 + xs2.T - 2.0 * dot, 0.0)

    inf = jnp.float32(np.inf)
    v1 = jnp.min(d, axis=1, keepdims=True)
    w2 = jnp.where(d == v1, inf, d)
    v2 = jnp.min(w2, axis=1, keepdims=True)
    w3 = jnp.where(w2 == v2, inf, w2)
    v3 = jnp.min(w3, axis=1, keepdims=True)
    wraw = jnp.where(d <= v3, 1.0 / (d + 1e-8), 0.0)
    norm = jnp.sum(wraw, axis=1, keepdims=True)
    wmat = wraw * (1.0 / norm)

    interp = jnp.dot(wmat, fs, preferred_element_type=jnp.float32,
                     precision=jax.lax.Precision.HIGHEST)
    CT = ft.shape[1]
    x1 = (jnp.dot(ft, w1t_ref[:CT, :], preferred_element_type=jnp.float32)
          + jnp.dot(interp, w1t_ref[CT:, :], preferred_element_type=jnp.float32))
    x1_ref[0] = x1

    @pl.when((pl.program_id(0) == 0) & (pl.program_id(1) == 0))
    def _init():
        stats_ref[...] = jnp.zeros_like(stats_ref)

    stats_ref[0, :] += jnp.sum(x1, axis=0)
    stats_ref[1, :] += jnp.sum(x1 * x1, axis=0)


def _mlp_kernel(x_ref, a_ref, c_ref, w2t_ref, x2_ref, stats_ref):
    y = jnp.maximum(x_ref[...] * a_ref[...] + c_ref[...], 0.0)
    x2 = jnp.dot(y, w2t_ref[...], preferred_element_type=jnp.float32)
    x2_ref[...] = x2

    @pl.when(pl.program_id(0) == 0)
    def _init():
        stats_ref[...] = jnp.zeros_like(stats_ref)

    stats_ref[0, :] += jnp.sum(x2, axis=0)
    stats_ref[1, :] += jnp.sum(x2 * x2, axis=0)


def _bnrelu_kernel(x_ref, a_ref, c_ref, o_ref):
    o_ref[...] = jnp.maximum(x_ref[...] * a_ref[...] + c_ref[...], 0.0)


def _bn_coeffs(stats, g, b, cnt):
    mean = stats[0] / cnt
    var = stats[1] / cnt - mean * mean
    rstd = jax.lax.rsqrt(var + 1e-5)
    a = (g * rstd)[None, :]
    c = (b - g * rstd * mean)[None, :]
    return a, c


def kernel(xyz_target, xyz_source, feat_target, feat_source, W1, g1, b1, W2, g2, b2):
    B, N, _ = xyz_target.shape
    S = xyz_source.shape[1]
    CT = feat_target.shape[2]
    CS = feat_source.shape[2]
    M1 = W1.shape[0]
    M2 = W2.shape[0]
    NB = min(512, N)
    gN = N // NB

    x1, stats1 = pl.pallas_call(
        _fp_kernel,
        grid=(B, gN),
        in_specs=[
            pl.BlockSpec((1, NB, 3), lambda b, n: (b, n, 0)),
            pl.BlockSpec((1, S, 3), lambda b, n: (b, 0, 0)),
            pl.BlockSpec((1, NB, CT), lambda b, n: (b, n, 0)),
            pl.BlockSpec((1, S, CS), lambda b, n: (b, 0, 0)),
            pl.BlockSpec((CT + CS, M1), lambda b, n: (0, 0)),
        ],
        out_specs=[
            pl.BlockSpec((1, NB, M1), lambda b, n: (b, n, 0)),
            pl.BlockSpec((2, M1), lambda b, n: (0, 0)),
        ],
        out_shape=[
            jax.ShapeDtypeStruct((B, N, M1), jnp.float32),
            jax.ShapeDtypeStruct((2, M1), jnp.float32),
        ],
    )(xyz_target, xyz_source, feat_target, feat_source, W1.T)

    cnt = jnp.float32(B * N)
    a1, c1 = _bn_coeffs(stats1, g1, b1, cnt)

    xf = x1.reshape(B * N, M1)
    NB2 = min(2048, B * N)
    g2n = (B * N) // NB2
    x2, stats2 = pl.pallas_call(
        _mlp_kernel,
        grid=(g2n,),
        in_specs=[
            pl.BlockSpec((NB2, M1), lambda i: (i, 0)),
            pl.BlockSpec((1, M1), lambda i: (0, 0)),
            pl.BlockSpec((1, M1), lambda i: (0, 0)),
            pl.BlockSpec((M1, M2), lambda i: (0, 0)),
        ],
        out_specs=[
            pl.BlockSpec((NB2, M2), lambda i: (i, 0)),
            pl.BlockSpec((2, M2), lambda i: (0, 0)),
        ],
        out_shape=[
            jax.ShapeDtypeStruct((B * N, M2), jnp.float32),
            jax.ShapeDtypeStruct((2, M2), jnp.float32),
        ],
    )(xf, a1, c1, W2.T)

    a2, c2 = _bn_coeffs(stats2, g2, b2, cnt)

    out = pl.pallas_call(
        _bnrelu_kernel,
        grid=(g2n,),
        in_specs=[
            pl.BlockSpec((NB2, M2), lambda i: (i, 0)),
            pl.BlockSpec((1, M2), lambda i: (0, 0)),
            pl.BlockSpec((1, M2), lambda i: (0, 0)),
        ],
        out_specs=pl.BlockSpec((NB2, M2), lambda i: (i, 0)),
        out_shape=jax.ShapeDtypeStruct((B * N, M2), jnp.float32),
    )(x2, a2, c2)
    return out.reshape(B, N, M2)


# speedup vs baseline: 1.4519x; 1.4519x over previous
import jax
import jax.numpy as jnp
import numpy as np
from jax.experimental import pallas as pl


def _fp_kernel(xt_ref, xs_ref, ft_ref, fs_ref, w1t_ref, x1_ref, stats_ref):
    S = xs_ref.shape[1]
    xt = xt_ref[0]
    xs = xs_ref[0]
    ft = ft_ref[0]
    fs = fs_ref[0]
    xt2 = jnp.sum(xt * xt, axis=1, keepdims=True)
    xs2 = jnp.sum(xs * xs, axis=1, keepdims=True)
    dot = jnp.dot(xt, xs.T, preferred_element_type=jnp.float32)
    d = jnp.maximum(xt2 + xs2.T - 2.0 * dot, 0.0)

    inf = jnp.float32(np.inf)
    v1 = jnp.min(d, axis=1, keepdims=True)
    w2 = jnp.where(d == v1, inf, d)
    v2 = jnp.min(w2, axis=1, keepdims=True)
    w3 = jnp.where(w2 == v2, inf, w2)
    v3 = jnp.min(w3, axis=1, keepdims=True)
    wraw = jnp.where(d <= v3, 1.0 / (d + 1e-8), 0.0)
    norm = jnp.sum(wraw, axis=1, keepdims=True)
    wmat = wraw * (1.0 / norm)

    interp = jnp.dot(wmat, fs, preferred_element_type=jnp.float32)
    CT = ft.shape[1]
    x1 = (jnp.dot(ft, w1t_ref[:CT, :], preferred_element_type=jnp.float32)
          + jnp.dot(interp, w1t_ref[CT:, :], preferred_element_type=jnp.float32))
    x1_ref[0] = x1

    @pl.when((pl.program_id(0) == 0) & (pl.program_id(1) == 0))
    def _init():
        stats_ref[...] = jnp.zeros_like(stats_ref)

    stats_ref[0, :] += jnp.sum(x1, axis=0)
    stats_ref[1, :] += jnp.sum(x1 * x1, axis=0)


def _mlp_kernel(x_ref, a_ref, c_ref, w2t_ref, x2_ref, stats_ref):
    y = jnp.maximum(x_ref[...] * a_ref[...] + c_ref[...], 0.0)
    x2 = jnp.dot(y, w2t_ref[...], preferred_element_type=jnp.float32)
    x2_ref[...] = x2

    @pl.when(pl.program_id(0) == 0)
    def _init():
        stats_ref[...] = jnp.zeros_like(stats_ref)

    stats_ref[0, :] += jnp.sum(x2, axis=0)
    stats_ref[1, :] += jnp.sum(x2 * x2, axis=0)


def _bnrelu_kernel(x_ref, a_ref, c_ref, o_ref):
    o_ref[...] = jnp.maximum(x_ref[...] * a_ref[...] + c_ref[...], 0.0)


def _bn_coeffs(stats, g, b, cnt):
    mean = stats[0] / cnt
    var = stats[1] / cnt - mean * mean
    rstd = jax.lax.rsqrt(var + 1e-5)
    a = (g * rstd)[None, :]
    c = (b - g * rstd * mean)[None, :]
    return a, c


def kernel(xyz_target, xyz_source, feat_target, feat_source, W1, g1, b1, W2, g2, b2):
    B, N, _ = xyz_target.shape
    S = xyz_source.shape[1]
    CT = feat_target.shape[2]
    CS = feat_source.shape[2]
    M1 = W1.shape[0]
    M2 = W2.shape[0]
    NB = min(512, N)
    gN = N // NB

    x1, stats1 = pl.pallas_call(
        _fp_kernel,
        grid=(B, gN),
        in_specs=[
            pl.BlockSpec((1, NB, 3), lambda b, n: (b, n, 0)),
            pl.BlockSpec((1, S, 3), lambda b, n: (b, 0, 0)),
            pl.BlockSpec((1, NB, CT), lambda b, n: (b, n, 0)),
            pl.BlockSpec((1, S, CS), lambda b, n: (b, 0, 0)),
            pl.BlockSpec((CT + CS, M1), lambda b, n: (0, 0)),
        ],
        out_specs=[
            pl.BlockSpec((1, NB, M1), lambda b, n: (b, n, 0)),
            pl.BlockSpec((2, M1), lambda b, n: (0, 0)),
        ],
        out_shape=[
            jax.ShapeDtypeStruct((B, N, M1), jnp.float32),
            jax.ShapeDtypeStruct((2, M1), jnp.float32),
        ],
    )(xyz_target, xyz_source, feat_target, feat_source, W1.T)

    cnt = jnp.float32(B * N)
    a1, c1 = _bn_coeffs(stats1, g1, b1, cnt)

    xf = x1.reshape(B * N, M1)
    NB2 = min(2048, B * N)
    g2n = (B * N) // NB2
    x2, stats2 = pl.pallas_call(
        _mlp_kernel,
        grid=(g2n,),
        in_specs=[
            pl.BlockSpec((NB2, M1), lambda i: (i, 0)),
            pl.BlockSpec((1, M1), lambda i: (0, 0)),
            pl.BlockSpec((1, M1), lambda i: (0, 0)),
            pl.BlockSpec((M1, M2), lambda i: (0, 0)),
        ],
        out_specs=[
            pl.BlockSpec((NB2, M2), lambda i: (i, 0)),
            pl.BlockSpec((2, M2), lambda i: (0, 0)),
        ],
        out_shape=[
            jax.ShapeDtypeStruct((B * N, M2), jnp.float32),
            jax.ShapeDtypeStruct((2, M2), jnp.float32),
        ],
    )(xf, a1, c1, W2.T)

    a2, c2 = _bn_coeffs(stats2, g2, b2, cnt)

    out = pl.pallas_call(
        _bnrelu_kernel,
        grid=(g2n,),
        in_specs=[
            pl.BlockSpec((NB2, M2), lambda i: (i, 0)),
            pl.BlockSpec((1, M2), lambda i: (0, 0)),
            pl.BlockSpec((1, M2), lambda i: (0, 0)),
        ],
        out_specs=pl.BlockSpec((NB2, M2), lambda i: (i, 0)),
        out_shape=jax.ShapeDtypeStruct((B * N, M2), jnp.float32),
    )(x2, a2, c2)
    return out.reshape(B, N, M2)
